# Initial kernel scaffold; baseline (speedup 1.0000x reference)
import jax
import jax.numpy as jnp
from jax.experimental import pallas as pl


def kernel(input, weights):
    raise NotImplementedError("write your pallas kernel here")


# speedup vs baseline: 4.5002x; 4.5002x over previous
import functools

import jax
import jax.numpy as jnp
from jax import lax
from jax.experimental import pallas as pl
from jax.experimental.pallas import tpu as pltpu
from jax.experimental.pallas import tpu_sc as plsc

_BATCH = 4096
_HIST = 50
_D = 64
_B = _BATCH * _HIST
_NC = 2
_NS = 16
_NW = _NC * _NS
_B_PER_W = _B // _NW
_CHUNK = 640
_NCHUNK = _B_PER_W // _CHUNK


def _emb_body(idx_hbm, table_hbm, out_hbm, idx_v, rows_v, sem):
  wid = lax.axis_index("s") * _NC + lax.axis_index("c")
  base = wid * _B_PER_W

  def step(i, carry):
    off = base + i * _CHUNK
    pltpu.sync_copy(idx_hbm.at[pl.ds(off, _CHUNK)], idx_v)
    pltpu.async_copy(table_hbm.at[idx_v], rows_v, sem).wait()
    pltpu.sync_copy(rows_v, out_hbm.at[pl.ds(off, _CHUNK)])
    return carry

  lax.fori_loop(0, _NCHUNK, step, 0)


_emb_call = pl.kernel(
    _emb_body,
    out_type=jax.ShapeDtypeStruct((_B, _D), jnp.float32),
    mesh=plsc.VectorSubcoreMesh(core_axis_name="c", subcore_axis_name="s"),
    scratch_types=[
        pltpu.VMEM((_CHUNK,), jnp.int32),
        pltpu.VMEM((_CHUNK, _D), jnp.float32),
        pltpu.SemaphoreType.DMA,
    ],
    compiler_params=pltpu.CompilerParams(use_tc_tiling_on_sc=False),
)


@jax.jit
def kernel(input, weights):
  idx = input.reshape(_B).astype(jnp.int32)
  out = _emb_call(idx, weights)
  return out.reshape(_BATCH, _HIST, _D)


# speedup vs baseline: 4.6079x; 1.0239x over previous
import functools

import jax
import jax.numpy as jnp
from jax import lax
from jax.experimental import pallas as pl
from jax.experimental.pallas import tpu as pltpu
from jax.experimental.pallas import tpu_sc as plsc

_BATCH = 4096
_HIST = 50
_D = 64
_B = _BATCH * _HIST
_NC = 2
_NS = 16
_NW = _NC * _NS
_B_PER_W = _B // _NW
_CHUNK = 640
_NCHUNK = _B_PER_W // _CHUNK


def _emb_body(idx_hbm, table_hbm, out_hbm, idx_v, rows_a, rows_b,
              gsem_a, gsem_b, wsem_a, wsem_b):
  wid = lax.axis_index("s") * _NC + lax.axis_index("c")
  base = wid * _B_PER_W
  rows = (rows_a, rows_b)
  gsem = (gsem_a, gsem_b)
  wsem = (wsem_a, wsem_b)

  pltpu.sync_copy(idx_hbm.at[pl.ds(base, _B_PER_W)], idx_v)

  def start_gather(i):
    b = i % 2
    return pltpu.async_copy(
        table_hbm.at[idx_v.at[pl.ds(i * _CHUNK, _CHUNK)]], rows[b], gsem[b])

  gathers = [None] * _NCHUNK
  writes = [None] * _NCHUNK
  gathers[0] = start_gather(0)
  for i in range(_NCHUNK):
    b = i % 2
    gathers[i].wait()
    if i + 1 < _NCHUNK:
      if i >= 1:
        writes[i - 1].wait()
      gathers[i + 1] = start_gather(i + 1)
    writes[i] = pltpu.async_copy(
        rows[b], out_hbm.at[pl.ds(base + i * _CHUNK, _CHUNK)], wsem[b])
  writes[_NCHUNK - 2].wait()
  writes[_NCHUNK - 1].wait()


_emb_call = pl.kernel(
    _emb_body,
    out_type=jax.ShapeDtypeStruct((_B, _D), jnp.float32),
    mesh=plsc.VectorSubcoreMesh(core_axis_name="c", subcore_axis_name="s"),
    scratch_types=[
        pltpu.VMEM((_B_PER_W,), jnp.int32),
        pltpu.VMEM((_CHUNK, _D), jnp.float32),
        pltpu.VMEM((_CHUNK, _D), jnp.float32),
        pltpu.SemaphoreType.DMA,
        pltpu.SemaphoreType.DMA,
        pltpu.SemaphoreType.DMA,
        pltpu.SemaphoreType.DMA,
    ],
    compiler_params=pltpu.CompilerParams(use_tc_tiling_on_sc=False),
)


@jax.jit
def kernel(input, weights):
  idx = input.reshape(_B).astype(jnp.int32)
  out = _emb_call(idx, weights)
  return out.reshape(_BATCH, _HIST, _D)
